# baseline (speedup 1.0000x reference)
import functools

import jax
import jax.numpy as jnp
from jax import lax
from jax.experimental import pallas as pl
from jax.experimental.pallas import tpu as pltpu
from jax.experimental.pallas import tpu_sc as plsc

_NC = 2
_NS = 16
_NW = _NC * _NS
_L = 16

_B = 16384
_C = 6
_D = 32

_BPW = _B // _NW
_OPW = _BPW * _C
_GROUPS = _OPW // _L


def _sc_kernel_body(target_hbm, ctx_hbm, ttab_hbm, ctab_hbm, out_hbm,
                    tidx_v, cidx_v, w_rows, c_rows, out_v, sem):
  wid = lax.axis_index("s") * _NC + lax.axis_index("c")
  bbase = wid * _BPW
  obase = wid * _OPW

  pltpu.sync_copy(target_hbm.at[pl.ds(bbase, _BPW)], tidx_v)
  pltpu.sync_copy(ctx_hbm.at[pl.ds(obase, _OPW)], cidx_v)

  cp_w = pltpu.async_copy(ttab_hbm.at[tidx_v], w_rows, sem)
  cp_c = pltpu.async_copy(ctab_hbm.at[cidx_v], c_rows, sem)
  cp_w.wait()
  cp_c.wait()

  it = lax.iota(jnp.int32, _L)

  def group(g, carry):
    k = g * _L + it
    crow = k
    wrow = k // _C
    acc = jnp.zeros((_L,), jnp.float32)
    for d in range(_D):
      col = jnp.full((_L,), d, jnp.int32)
      cv = plsc.load_gather(c_rows, [crow, col])
      wv = plsc.load_gather(w_rows, [wrow, col])
      acc = acc + cv * wv
    out_v[pl.ds(g * _L, _L)] = acc
    return carry

  lax.fori_loop(0, _GROUPS, group, 0, unroll=False)

  pltpu.sync_copy(out_v, out_hbm.at[pl.ds(obase, _OPW)])


@functools.partial(jax.jit, static_argnames=())
def kernel(target, context, target_table, context_table):
  ctx_flat = context.reshape(-1)
  mesh = plsc.VectorSubcoreMesh(core_axis_name="c", subcore_axis_name="s")
  out_flat = pl.kernel(
      _sc_kernel_body,
      out_type=jax.ShapeDtypeStruct((_B * _C,), jnp.float32),
      mesh=mesh,
      compiler_params=pltpu.CompilerParams(
          needs_layout_passes=False, use_tc_tiling_on_sc=False),
      scratch_types=[
          pltpu.VMEM((_BPW,), jnp.int32),
          pltpu.VMEM((_OPW,), jnp.int32),
          pltpu.VMEM((_BPW, _D), jnp.float32),
          pltpu.VMEM((_OPW, _D), jnp.float32),
          pltpu.VMEM((_OPW,), jnp.float32),
          pltpu.SemaphoreType.DMA,
      ],
  )(target, ctx_flat, target_table, context_table)
  return out_flat.reshape(_B, _C)


# speedup vs baseline: 2.6215x; 2.6215x over previous
import functools

import jax
import jax.numpy as jnp
from jax import lax
from jax.experimental import pallas as pl
from jax.experimental.pallas import tpu as pltpu
from jax.experimental.pallas import tpu_sc as plsc

_NC = 2
_NS = 16
_L = 16

_B = 16384
_C = 6
_D = 32
_V = 1000000

_DPC = _D // _NC
_WPT = _B // _NS
_KPT = _WPT * _C
_CHK = 512
_NCHK = _KPT // _CHK
_CGRP = _CHK // _L

_OPW = (_B * _C) // (_NC * _NS)
_K2CH = 768
_K2N = _OPW // _K2CH


def _gather_prod_body(tgt_hbm, ctx_hbm, ttabT_hbm, ctabT_hbm, pprod_hbm,
                      widx, wslab, cidx2, cslab2, buf_t, buf_c,
                      sem_t, sem_c, gsem_w, gsem2, isem2, osem2):
  cid = lax.axis_index("c")
  sid = lax.axis_index("s")

  pltpu.sync_copy(tgt_hbm.at[pl.ds(sid * _WPT, _WPT)], widx)

  for c in range(_NC):
    @pl.when((sid == 0) & (cid == c))
    def _(c=c):
      pltpu.async_copy(ttabT_hbm.at[c * _DPC], buf_t, sem_t)

  it = lax.iota(jnp.int32, _L)

  def cidx_fetch(ch, b):
    kbase = sid * _KPT + ch * _CHK
    return pltpu.async_copy(ctx_hbm.at[pl.ds(kbase, _CHK)], cidx2[b], isem2[b])

  def gather_fire(ch, b):
    return pltpu.async_copy(buf_c.at[cidx2[b]], cslab2[b], gsem2[b])

  def compute_and_emit(dl, ch, b):
    def group(g, carry):
      sl = pl.ds(g * _L, _L)
      k = ch * _CHK + g * _L + it
      wrow = k // _C
      wv = plsc.load_gather(wslab, [wrow])
      cslab2[b][sl] = cslab2[b][sl] * wv
      return carry

    lax.fori_loop(0, _CGRP, group, 0)
    kbase = sid * _KPT + ch * _CHK
    for c in range(_NC):
      @pl.when(cid == c)
      def _(c=c):
        pltpu.async_copy(
            cslab2[b], pprod_hbm.at[c * _DPC + dl, pl.ds(kbase, _CHK)],
            osem2[b])

  for dl in range(_DPC):
    for c in range(_NC):
      @pl.when((sid == 0) & (cid == c))
      def _(c=c):
        pltpu.make_async_copy(ttabT_hbm.at[c * _DPC + dl], buf_t, sem_t).wait()
    plsc.subcore_barrier()

    pltpu.async_copy(buf_t.at[widx], wslab, gsem_w).wait()

    for c in range(_NC):
      @pl.when((sid == 0) & (cid == c))
      def _(c=c):
        pltpu.async_copy(ctabT_hbm.at[c * _DPC + dl], buf_c, sem_c)

    plsc.subcore_barrier()

    if dl + 1 < _DPC:
      for c in range(_NC):
        @pl.when((sid == 0) & (cid == c))
        def _(c=c):
          pltpu.async_copy(ttabT_hbm.at[c * _DPC + dl + 1], buf_t, sem_t)

    for c in range(_NC):
      @pl.when((sid == 0) & (cid == c))
      def _(c=c):
        pltpu.make_async_copy(ctabT_hbm.at[c * _DPC + dl], buf_c, sem_c).wait()
    plsc.subcore_barrier()

    cidx_fetch(0, 0).wait()
    gather_fire(0, 0)
    cidx_fetch(1, 1)
    for ch in range(1, _NCHK):
      b = ch % 2
      pb = 1 - b
      pltpu.make_async_copy(
          ctx_hbm.at[pl.ds(sid * _KPT + ch * _CHK, _CHK)], cidx2[b],
          isem2[b]).wait()
      if ch >= 2:
        pltpu.make_async_copy(
            cslab2[b],
            pprod_hbm.at[0, pl.ds(sid * _KPT + (ch - 2) * _CHK, _CHK)],
            osem2[b]).wait()
      gather_fire(ch, b)
      pltpu.make_async_copy(buf_c.at[cidx2[pb]], cslab2[pb], gsem2[pb]).wait()
      if ch + 1 < _NCHK:
        cidx_fetch(ch + 1, pb)
      compute_and_emit(dl, ch - 1, pb)
    b = (_NCHK - 1) % 2
    pltpu.make_async_copy(buf_c.at[cidx2[b]], cslab2[b], gsem2[b]).wait()
    compute_and_emit(dl, _NCHK - 1, b)
    for b in range(2):
      pltpu.make_async_copy(
          cslab2[b], pprod_hbm.at[0, pl.ds(sid * _KPT, _CHK)], osem2[b]).wait()
    plsc.subcore_barrier()


def _rowsum_body(p_hbm, out_hbm, obuf, *inbufs):
  cid = lax.axis_index("c")
  sid = lax.axis_index("s")
  wid = sid * _NC + cid
  base = wid * _OPW

  for ch in range(_K2N):
    off = base + ch * _K2CH
    for r in range(_D):
      pltpu.sync_copy(p_hbm.at[r, pl.ds(off, _K2CH)], inbufs[r])

    def group(g, carry):
      sl = pl.ds(g * _L, _L)
      acc = inbufs[0][sl]
      for r in range(1, _D):
        acc = acc + inbufs[r][sl]
      obuf[sl] = acc
      return carry

    lax.fori_loop(0, _K2CH // _L, group, 0)
    pltpu.sync_copy(obuf, out_hbm.at[pl.ds(off, _K2CH)])


@functools.partial(jax.jit, static_argnames=())
def kernel(target, context, target_table, context_table):
  ctx_flat = context.reshape(-1)
  mesh = plsc.VectorSubcoreMesh(core_axis_name="c", subcore_axis_name="s")
  pprod = pl.kernel(
      _gather_prod_body,
      out_type=jax.ShapeDtypeStruct((_D, _B * _C), jnp.float32),
      mesh=mesh,
      compiler_params=pltpu.CompilerParams(needs_layout_passes=False),
      scratch_types=[
          pltpu.VMEM((_WPT,), jnp.int32),
          pltpu.VMEM((_WPT,), jnp.float32),
          [pltpu.VMEM((_CHK,), jnp.int32) for _ in range(2)],
          [pltpu.VMEM((_CHK,), jnp.float32) for _ in range(2)],
          pltpu.VMEM_SHARED((_V,), jnp.float32),
          pltpu.VMEM_SHARED((_V,), jnp.float32),
          pltpu.SemaphoreType.DMA,
          pltpu.SemaphoreType.DMA,
          pltpu.SemaphoreType.DMA,
          [pltpu.SemaphoreType.DMA for _ in range(2)],
          [pltpu.SemaphoreType.DMA for _ in range(2)],
          [pltpu.SemaphoreType.DMA for _ in range(2)],
      ],
  )(target, ctx_flat, target_table.T, context_table.T)

  out_flat = pl.kernel(
      _rowsum_body,
      out_type=jax.ShapeDtypeStruct((_B * _C,), jnp.float32),
      mesh=mesh,
      compiler_params=pltpu.CompilerParams(needs_layout_passes=False),
      scratch_types=[pltpu.VMEM((_K2CH,), jnp.float32)]
        + [pltpu.VMEM((_K2CH,), jnp.float32) for _ in range(_D)],
  )(pprod)
  return out_flat.reshape(_B, _C)


# speedup vs baseline: 3.1461x; 1.2001x over previous
import functools

import jax
import jax.numpy as jnp
from jax import lax
from jax.experimental import pallas as pl
from jax.experimental.pallas import tpu as pltpu
from jax.experimental.pallas import tpu_sc as plsc

_NC = 2
_NS = 16
_L = 16

_B = 16384
_C = 6
_D = 32
_V = 1000000

_DPC = _D // _NC
_WPT = _B // _NS
_KPT = _WPT * _C
_CHK = 512
_NCHK = _KPT // _CHK
_CGRP = _CHK // _L

_OPW = (_B * _C) // (_NC * _NS)
_K2CH = 768
_K2N = _OPW // _K2CH


def _gather_prod_body(tgt_hbm, ctx_hbm, ttabT_hbm, ctabT_hbm, pprod_hbm,
                      widx, wslab, cidx2, cslab2, buf_t, buf_c,
                      sem_t, sem_c, gsem_w, gsem2, isem2, osem2):
  cid = lax.axis_index("c")
  sid = lax.axis_index("s")

  pltpu.sync_copy(tgt_hbm.at[pl.ds(sid * _WPT, _WPT)], widx)

  for c in range(_NC):
    @pl.when((sid == 0) & (cid == c))
    def _(c=c):
      pltpu.async_copy(ttabT_hbm.at[c * _DPC], buf_t, sem_t)

  it = lax.iota(jnp.int32, _L)

  def cidx_fetch(ch, b):
    kbase = sid * _KPT + ch * _CHK
    return pltpu.async_copy(ctx_hbm.at[pl.ds(kbase, _CHK)], cidx2[b], isem2[b])

  def gather_fire(ch, b):
    return pltpu.async_copy(buf_c.at[cidx2[b]], cslab2[b], gsem2[b])

  def compute_and_emit(dl, ch, b):
    def group(g, carry):
      sl = pl.ds(g * _L, _L)
      k = ch * _CHK + g * _L + it
      wrow = k // _C
      wv = plsc.load_gather(wslab, [wrow])
      cslab2[b][sl] = cslab2[b][sl] * wv
      return carry

    lax.fori_loop(0, _CGRP, group, 0)
    kbase = sid * _KPT + ch * _CHK
    for c in range(_NC):
      @pl.when(cid == c)
      def _(c=c):
        pltpu.async_copy(
            cslab2[b], pprod_hbm.at[c * _DPC + dl, pl.ds(kbase, _CHK)],
            osem2[b])

  for dl in range(_DPC):
    for c in range(_NC):
      @pl.when((sid == 0) & (cid == c))
      def _(c=c):
        pltpu.make_async_copy(ttabT_hbm.at[c * _DPC + dl], buf_t, sem_t).wait()
    plsc.subcore_barrier()

    pltpu.async_copy(buf_t.at[widx], wslab, gsem_w).wait()

    for c in range(_NC):
      @pl.when((sid == 0) & (cid == c))
      def _(c=c):
        pltpu.async_copy(ctabT_hbm.at[c * _DPC + dl], buf_c, sem_c)

    plsc.subcore_barrier()

    if dl + 1 < _DPC:
      for c in range(_NC):
        @pl.when((sid == 0) & (cid == c))
        def _(c=c):
          pltpu.async_copy(ttabT_hbm.at[c * _DPC + dl + 1], buf_t, sem_t)

    for c in range(_NC):
      @pl.when((sid == 0) & (cid == c))
      def _(c=c):
        pltpu.make_async_copy(ctabT_hbm.at[c * _DPC + dl], buf_c, sem_c).wait()
    plsc.subcore_barrier()

    cidx_fetch(0, 0).wait()
    gather_fire(0, 0)
    cidx_fetch(1, 1)
    for ch in range(1, _NCHK):
      b = ch % 2
      pb = 1 - b
      pltpu.make_async_copy(
          ctx_hbm.at[pl.ds(sid * _KPT + ch * _CHK, _CHK)], cidx2[b],
          isem2[b]).wait()
      if ch >= 2:
        pltpu.make_async_copy(
            cslab2[b],
            pprod_hbm.at[0, pl.ds(sid * _KPT + (ch - 2) * _CHK, _CHK)],
            osem2[b]).wait()
      gather_fire(ch, b)
      pltpu.make_async_copy(buf_c.at[cidx2[pb]], cslab2[pb], gsem2[pb]).wait()
      if ch + 1 < _NCHK:
        cidx_fetch(ch + 1, pb)
      compute_and_emit(dl, ch - 1, pb)
    b = (_NCHK - 1) % 2
    pltpu.make_async_copy(buf_c.at[cidx2[b]], cslab2[b], gsem2[b]).wait()
    compute_and_emit(dl, _NCHK - 1, b)
    for b in range(2):
      pltpu.make_async_copy(
          cslab2[b], pprod_hbm.at[0, pl.ds(sid * _KPT, _CHK)], osem2[b]).wait()
    plsc.subcore_barrier()


def _rowsum_body(p_hbm, out_hbm, ksem, obuf, *inbufs):
  cid = lax.axis_index("c")
  sid = lax.axis_index("s")
  wid = sid * _NC + cid
  base = wid * _OPW

  for ch in range(_K2N):
    off = base + ch * _K2CH
    for r in range(_D):
      pltpu.async_copy(p_hbm.at[r, pl.ds(off, _K2CH)], inbufs[r], ksem)
    for r in range(_D):
      pltpu.make_async_copy(p_hbm.at[r, pl.ds(off, _K2CH)], inbufs[r],
                            ksem).wait()

    def group(g, carry):
      sl = pl.ds(g * _L, _L)
      acc = inbufs[0][sl]
      for r in range(1, _D):
        acc = acc + inbufs[r][sl]
      obuf[sl] = acc
      return carry

    lax.fori_loop(0, _K2CH // _L, group, 0)
    pltpu.sync_copy(obuf, out_hbm.at[pl.ds(off, _K2CH)])


@functools.partial(jax.jit, static_argnames=())
def kernel(target, context, target_table, context_table):
  ctx_flat = context.reshape(-1)
  mesh = plsc.VectorSubcoreMesh(core_axis_name="c", subcore_axis_name="s")
  pprod = pl.kernel(
      _gather_prod_body,
      out_type=jax.ShapeDtypeStruct((_D, _B * _C), jnp.float32),
      mesh=mesh,
      compiler_params=pltpu.CompilerParams(needs_layout_passes=False),
      scratch_types=[
          pltpu.VMEM((_WPT,), jnp.int32),
          pltpu.VMEM((_WPT,), jnp.float32),
          [pltpu.VMEM((_CHK,), jnp.int32) for _ in range(2)],
          [pltpu.VMEM((_CHK,), jnp.float32) for _ in range(2)],
          pltpu.VMEM_SHARED((_V,), jnp.float32),
          pltpu.VMEM_SHARED((_V,), jnp.float32),
          pltpu.SemaphoreType.DMA,
          pltpu.SemaphoreType.DMA,
          pltpu.SemaphoreType.DMA,
          [pltpu.SemaphoreType.DMA for _ in range(2)],
          [pltpu.SemaphoreType.DMA for _ in range(2)],
          [pltpu.SemaphoreType.DMA for _ in range(2)],
      ],
  )(target, ctx_flat, target_table.T, context_table.T)

  out_flat = pl.kernel(
      _rowsum_body,
      out_type=jax.ShapeDtypeStruct((_B * _C,), jnp.float32),
      mesh=mesh,
      compiler_params=pltpu.CompilerParams(needs_layout_passes=False),
      scratch_types=[pltpu.SemaphoreType.DMA, pltpu.VMEM((_K2CH,), jnp.float32)]
        + [pltpu.VMEM((_K2CH,), jnp.float32) for _ in range(_D)],
  )(pprod)
  return out_flat.reshape(_B, _C)


# speedup vs baseline: 3.4207x; 1.0873x over previous
import functools

import jax
import jax.numpy as jnp
from jax import lax
from jax.experimental import pallas as pl
from jax.experimental.pallas import tpu as pltpu
from jax.experimental.pallas import tpu_sc as plsc

_NC = 2
_NS = 16
_L = 16

_B = 16384
_C = 6
_D = 32
_V = 1000000

_DPC = _D // _NC
_WPT = _B // _NS
_KPT = _WPT * _C
_CHK = 512
_NCHK = _KPT // _CHK
_CGRP = _CHK // _L

_OPW = (_B * _C) // (_NC * _NS)
_K2CH = 1536
_K2N = _OPW // _K2CH


def _gather_prod_body(tgt_hbm, ctx_hbm, ttabT_hbm, ctabT_hbm, pprod_hbm,
                      widx, wslab, cidx2, cslab2, buf_t, buf_c,
                      sem_t, sem_c, gsem_w, gsem2, isem2, osem2):
  cid = lax.axis_index("c")
  sid = lax.axis_index("s")

  pltpu.sync_copy(tgt_hbm.at[pl.ds(sid * _WPT, _WPT)], widx)

  for c in range(_NC):
    @pl.when((sid == 0) & (cid == c))
    def _(c=c):
      pltpu.async_copy(ttabT_hbm.at[c * _DPC], buf_t, sem_t)

  it = lax.iota(jnp.int32, _L)

  def cidx_fetch(ch, b):
    kbase = sid * _KPT + ch * _CHK
    return pltpu.async_copy(ctx_hbm.at[pl.ds(kbase, _CHK)], cidx2[b], isem2[b])

  def gather_fire(ch, b):
    return pltpu.async_copy(buf_c.at[cidx2[b]], cslab2[b], gsem2[b])

  def compute_and_emit(dl, ch, b):
    def group(g, carry):
      sl = pl.ds(g * _L, _L)
      k = ch * _CHK + g * _L + it
      wrow = k // _C
      wv = plsc.load_gather(wslab, [wrow])
      cslab2[b][sl] = cslab2[b][sl] * wv
      return carry

    lax.fori_loop(0, _CGRP, group, 0)
    kbase = sid * _KPT + ch * _CHK
    for c in range(_NC):
      @pl.when(cid == c)
      def _(c=c):
        pltpu.async_copy(
            cslab2[b], pprod_hbm.at[c * _DPC + dl, pl.ds(kbase, _CHK)],
            osem2[b])

  for dl in range(_DPC):
    for c in range(_NC):
      @pl.when((sid == 0) & (cid == c))
      def _(c=c):
        pltpu.make_async_copy(ttabT_hbm.at[c * _DPC + dl], buf_t, sem_t).wait()
    plsc.subcore_barrier()

    for c in range(_NC):
      @pl.when((sid == 0) & (cid == c))
      def _(c=c):
        pltpu.async_copy(ctabT_hbm.at[c * _DPC + dl], buf_c, sem_c)

    pltpu.async_copy(buf_t.at[widx], wslab, gsem_w).wait()

    for c in range(_NC):
      @pl.when((sid == 0) & (cid == c))
      def _(c=c):
        pltpu.make_async_copy(ctabT_hbm.at[c * _DPC + dl], buf_c, sem_c).wait()
    plsc.subcore_barrier()

    if dl + 1 < _DPC:
      for c in range(_NC):
        @pl.when((sid == 0) & (cid == c))
        def _(c=c):
          pltpu.async_copy(ttabT_hbm.at[c * _DPC + dl + 1], buf_t, sem_t)

    cidx_fetch(0, 0).wait()
    gather_fire(0, 0)
    cidx_fetch(1, 1)
    for ch in range(1, _NCHK):
      b = ch % 2
      pb = 1 - b
      pltpu.make_async_copy(
          ctx_hbm.at[pl.ds(sid * _KPT + ch * _CHK, _CHK)], cidx2[b],
          isem2[b]).wait()
      if ch >= 2:
        pltpu.make_async_copy(
            cslab2[b],
            pprod_hbm.at[0, pl.ds(sid * _KPT + (ch - 2) * _CHK, _CHK)],
            osem2[b]).wait()
      gather_fire(ch, b)
      pltpu.make_async_copy(buf_c.at[cidx2[pb]], cslab2[pb], gsem2[pb]).wait()
      if ch + 1 < _NCHK:
        cidx_fetch(ch + 1, pb)
      compute_and_emit(dl, ch - 1, pb)
    b = (_NCHK - 1) % 2
    pltpu.make_async_copy(buf_c.at[cidx2[b]], cslab2[b], gsem2[b]).wait()
    compute_and_emit(dl, _NCHK - 1, b)
    for b in range(2):
      pltpu.make_async_copy(
          cslab2[b], pprod_hbm.at[0, pl.ds(sid * _KPT, _CHK)], osem2[b]).wait()
    plsc.subcore_barrier()


def _rowsum_body(p_hbm, out_hbm, ksem, obuf, *inbufs):
  cid = lax.axis_index("c")
  sid = lax.axis_index("s")
  wid = sid * _NC + cid
  base = wid * _OPW

  for ch in range(_K2N):
    off = base + ch * _K2CH
    for r in range(_D):
      pltpu.async_copy(p_hbm.at[r, pl.ds(off, _K2CH)], inbufs[r], ksem)
    for r in range(_D):
      pltpu.make_async_copy(p_hbm.at[r, pl.ds(off, _K2CH)], inbufs[r],
                            ksem).wait()

    def group(g, carry):
      sl = pl.ds(g * _L, _L)
      acc = inbufs[0][sl]
      for r in range(1, _D):
        acc = acc + inbufs[r][sl]
      obuf[sl] = acc
      return carry

    lax.fori_loop(0, _K2CH // _L, group, 0)
    pltpu.sync_copy(obuf, out_hbm.at[pl.ds(off, _K2CH)])


@functools.partial(jax.jit, static_argnames=())
def kernel(target, context, target_table, context_table):
  ctx_flat = context.reshape(-1)
  mesh = plsc.VectorSubcoreMesh(core_axis_name="c", subcore_axis_name="s")
  pprod = pl.kernel(
      _gather_prod_body,
      out_type=jax.ShapeDtypeStruct((_D, _B * _C), jnp.float32),
      mesh=mesh,
      compiler_params=pltpu.CompilerParams(needs_layout_passes=False),
      scratch_types=[
          pltpu.VMEM((_WPT,), jnp.int32),
          pltpu.VMEM((_WPT,), jnp.float32),
          [pltpu.VMEM((_CHK,), jnp.int32) for _ in range(2)],
          [pltpu.VMEM((_CHK,), jnp.float32) for _ in range(2)],
          pltpu.VMEM_SHARED((_V,), jnp.float32),
          pltpu.VMEM_SHARED((_V,), jnp.float32),
          pltpu.SemaphoreType.DMA,
          pltpu.SemaphoreType.DMA,
          pltpu.SemaphoreType.DMA,
          [pltpu.SemaphoreType.DMA for _ in range(2)],
          [pltpu.SemaphoreType.DMA for _ in range(2)],
          [pltpu.SemaphoreType.DMA for _ in range(2)],
      ],
  )(target, ctx_flat, target_table.T, context_table.T)

  out_flat = pl.kernel(
      _rowsum_body,
      out_type=jax.ShapeDtypeStruct((_B * _C,), jnp.float32),
      mesh=mesh,
      compiler_params=pltpu.CompilerParams(needs_layout_passes=False),
      scratch_types=[pltpu.SemaphoreType.DMA, pltpu.VMEM((_K2CH,), jnp.float32)]
        + [pltpu.VMEM((_K2CH,), jnp.float32) for _ in range(_D)],
  )(pprod)
  return out_flat.reshape(_B, _C)


# speedup vs baseline: 3.4407x; 1.0058x over previous
import functools

import jax
import jax.numpy as jnp
from jax import lax
from jax.experimental import pallas as pl
from jax.experimental.pallas import tpu as pltpu
from jax.experimental.pallas import tpu_sc as plsc

_NC = 2
_NS = 16
_L = 16

_B = 16384
_C = 6
_D = 32
_V = 1000000

_DPC = _D // _NC
_WPT = _B // _NS
_KPT = _WPT * _C
_CHK = 512
_NCHK = _KPT // _CHK
_CGRP = _CHK // _L

_OPW = (_B * _C) // (_NC * _NS)
_K2CH = 3072
_K2N = _OPW // _K2CH


def _gather_prod_body(tgt_hbm, ctx_hbm, ttabT_hbm, ctabT_hbm, pprod_hbm,
                      widx, wslab, cidx2, cslab2, buf_t, buf_c,
                      sem_t, sem_c, gsem_w, gsem2, isem2, osem2):
  cid = lax.axis_index("c")
  sid = lax.axis_index("s")

  pltpu.sync_copy(tgt_hbm.at[pl.ds(sid * _WPT, _WPT)], widx)

  for c in range(_NC):
    @pl.when((sid == 0) & (cid == c))
    def _(c=c):
      pltpu.async_copy(ttabT_hbm.at[c * _DPC], buf_t, sem_t)

  it = lax.iota(jnp.int32, _L)

  def cidx_fetch(ch, b):
    kbase = sid * _KPT + ch * _CHK
    return pltpu.async_copy(ctx_hbm.at[pl.ds(kbase, _CHK)], cidx2[b], isem2[b])

  def gather_fire(ch, b):
    return pltpu.async_copy(buf_c.at[cidx2[b]], cslab2[b], gsem2[b])

  def compute_and_emit(dl, ch, b):
    def group(g, carry):
      sl = pl.ds(g * _L, _L)
      k = ch * _CHK + g * _L + it
      wrow = k // _C
      wv = plsc.load_gather(wslab, [wrow])
      cslab2[b][sl] = cslab2[b][sl] * wv
      return carry

    lax.fori_loop(0, _CGRP, group, 0)
    kbase = sid * _KPT + ch * _CHK
    for c in range(_NC):
      @pl.when(cid == c)
      def _(c=c):
        pltpu.async_copy(
            cslab2[b], pprod_hbm.at[c * _DPC + dl, pl.ds(kbase, _CHK)],
            osem2[b])

  for dl in range(_DPC):
    for c in range(_NC):
      @pl.when((sid == 0) & (cid == c))
      def _(c=c):
        pltpu.make_async_copy(ttabT_hbm.at[c * _DPC + dl], buf_t, sem_t).wait()
    plsc.subcore_barrier()

    for c in range(_NC):
      @pl.when((sid == 0) & (cid == c))
      def _(c=c):
        pltpu.async_copy(ctabT_hbm.at[c * _DPC + dl], buf_c, sem_c)

    pltpu.async_copy(buf_t.at[widx], wslab, gsem_w).wait()

    for c in range(_NC):
      @pl.when((sid == 0) & (cid == c))
      def _(c=c):
        pltpu.make_async_copy(ctabT_hbm.at[c * _DPC + dl], buf_c, sem_c).wait()
    plsc.subcore_barrier()

    if dl + 1 < _DPC:
      for c in range(_NC):
        @pl.when((sid == 0) & (cid == c))
        def _(c=c):
          pltpu.async_copy(ttabT_hbm.at[c * _DPC + dl + 1], buf_t, sem_t)

    cidx_fetch(0, 0).wait()
    gather_fire(0, 0)
    cidx_fetch(1, 1)
    for ch in range(1, _NCHK):
      b = ch % 2
      pb = 1 - b
      pltpu.make_async_copy(
          ctx_hbm.at[pl.ds(sid * _KPT + ch * _CHK, _CHK)], cidx2[b],
          isem2[b]).wait()
      if ch >= 2:
        pltpu.make_async_copy(
            cslab2[b],
            pprod_hbm.at[0, pl.ds(sid * _KPT + (ch - 2) * _CHK, _CHK)],
            osem2[b]).wait()
      gather_fire(ch, b)
      pltpu.make_async_copy(buf_c.at[cidx2[pb]], cslab2[pb], gsem2[pb]).wait()
      if ch + 1 < _NCHK:
        cidx_fetch(ch + 1, pb)
      compute_and_emit(dl, ch - 1, pb)
    b = (_NCHK - 1) % 2
    pltpu.make_async_copy(buf_c.at[cidx2[b]], cslab2[b], gsem2[b]).wait()
    compute_and_emit(dl, _NCHK - 1, b)
    for b in range(2):
      pltpu.make_async_copy(
          cslab2[b], pprod_hbm.at[0, pl.ds(sid * _KPT, _CHK)], osem2[b]).wait()
    plsc.subcore_barrier()


def _rowsum_body(p_hbm, out_hbm, ksem, obuf, *inbufs):
  cid = lax.axis_index("c")
  sid = lax.axis_index("s")
  wid = sid * _NC + cid
  base = wid * _OPW

  for ch in range(_K2N):
    off = base + ch * _K2CH
    for r in range(_D):
      pltpu.async_copy(p_hbm.at[r, pl.ds(off, _K2CH)], inbufs[r], ksem)
    for r in range(_D):
      pltpu.make_async_copy(p_hbm.at[r, pl.ds(off, _K2CH)], inbufs[r],
                            ksem).wait()

    def group(g, carry):
      sl = pl.ds(g * _L, _L)
      acc = inbufs[0][sl]
      for r in range(1, _D):
        acc = acc + inbufs[r][sl]
      obuf[sl] = acc
      return carry

    lax.fori_loop(0, _K2CH // _L, group, 0)
    pltpu.sync_copy(obuf, out_hbm.at[pl.ds(off, _K2CH)])


@functools.partial(jax.jit, static_argnames=())
def kernel(target, context, target_table, context_table):
  ctx_flat = context.reshape(-1)
  mesh = plsc.VectorSubcoreMesh(core_axis_name="c", subcore_axis_name="s")
  pprod = pl.kernel(
      _gather_prod_body,
      out_type=jax.ShapeDtypeStruct((_D, _B * _C), jnp.float32),
      mesh=mesh,
      compiler_params=pltpu.CompilerParams(needs_layout_passes=False),
      scratch_types=[
          pltpu.VMEM((_WPT,), jnp.int32),
          pltpu.VMEM((_WPT,), jnp.float32),
          [pltpu.VMEM((_CHK,), jnp.int32) for _ in range(2)],
          [pltpu.VMEM((_CHK,), jnp.float32) for _ in range(2)],
          pltpu.VMEM_SHARED((_V,), jnp.float32),
          pltpu.VMEM_SHARED((_V,), jnp.float32),
          pltpu.SemaphoreType.DMA,
          pltpu.SemaphoreType.DMA,
          pltpu.SemaphoreType.DMA,
          [pltpu.SemaphoreType.DMA for _ in range(2)],
          [pltpu.SemaphoreType.DMA for _ in range(2)],
          [pltpu.SemaphoreType.DMA for _ in range(2)],
      ],
  )(target, ctx_flat, target_table.T, context_table.T)

  out_flat = pl.kernel(
      _rowsum_body,
      out_type=jax.ShapeDtypeStruct((_B * _C,), jnp.float32),
      mesh=mesh,
      compiler_params=pltpu.CompilerParams(needs_layout_passes=False),
      scratch_types=[pltpu.SemaphoreType.DMA, pltpu.VMEM((_K2CH,), jnp.float32)]
        + [pltpu.VMEM((_K2CH,), jnp.float32) for _ in range(_D)],
  )(pprod)
  return out_flat.reshape(_B, _C)
